# baseline (speedup 1.0000x reference)
import functools

import jax
import jax.numpy as jnp
from jax import lax
from jax.experimental import pallas as pl
from jax.experimental.pallas import tpu as pltpu
from jax.experimental.pallas import tpu_sc as plsc

VOCAB = 1000000
MAX_LEN = 200
EMBED_DIM = 64
BATCH = 4096

NUM_CORES = 2
NUM_SUBCORES = 16
NW = NUM_CORES * NUM_SUBCORES
BBLK = 128
NBC = BATCH // BBLK
N_BLOCKS = MAX_LEN * NBC
BLOCKS_PER_W = N_BLOCKS // NW
DG = EMBED_DIM // 16


def _build():
  mesh = plsc.VectorSubcoreMesh(core_axis_name="c", subcore_axis_name="s")

  @functools.partial(
      pl.kernel,
      mesh=mesh,
      compiler_params=pltpu.CompilerParams(use_tc_tiling_on_sc=False,
                                           needs_layout_passes=False),
      out_type=jax.ShapeDtypeStruct((MAX_LEN, 8, NBC, 8 * BBLK), jnp.float32),
      scratch_types=[
          pltpu.VMEM((BBLK,), jnp.int32),
          pltpu.VMEM((BBLK,), jnp.int32),
          pltpu.VMEM((BBLK, EMBED_DIM), jnp.float32),
          pltpu.VMEM((BBLK, EMBED_DIM), jnp.float32),
          pltpu.VMEM((EMBED_DIM * BBLK,), jnp.float32),
          pltpu.VMEM((EMBED_DIM * BBLK,), jnp.float32),
          pltpu.VMEM((MAX_LEN, EMBED_DIM), jnp.float32),
          pltpu.SemaphoreType.DMA,
          pltpu.SemaphoreType.DMA,
          pltpu.SemaphoreType.DMA,
          pltpu.SemaphoreType.DMA,
      ],
  )
  def emb_kernel(idxT_hbm, table_hbm, pos_hbm, out_hbm,
                 idx_a, idx_b, g_a, g_b, t_a, t_b, pos_v,
                 gsem_a, gsem_b, wsem_a, wsem_b):
    wid = lax.axis_index("s") * NUM_CORES + lax.axis_index("c")
    bid0 = wid * BLOCKS_PER_W

    pltpu.sync_copy(pos_hbm, pos_v)

    iota16 = lax.iota(jnp.int32, 16)
    rowvecs = [(dg * 16 + iota16) * BBLK for dg in range(DG)]

    bufs = ((idx_a, g_a, t_a, gsem_a, wsem_a),
            (idx_b, g_b, t_b, gsem_b, wsem_b))

    def fire(buf, bid):
      idx_v, gbuf, _, gsem, _ = buf
      l = bid // NBC
      c = bid - l * NBC
      pltpu.sync_copy(idxT_hbm.at[l, pl.ds(c * BBLK, BBLK)], idx_v)
      pltpu.async_copy(table_hbm.at[idx_v], gbuf, gsem)

    def wait_write(buf):
      _, _, tbuf, _, wsem = buf
      for g in range(8):
        pltpu.make_async_copy(tbuf.at[pl.ds(g * 1024, 1024)],
                              out_hbm.at[0, g, 0], wsem).wait()

    def process(buf, bid, first):
      idx_v, gbuf, tbuf, gsem, wsem = buf
      l = bid // NBC
      c = bid - l * NBC
      pltpu.make_async_copy(table_hbm.at[idx_v], gbuf, gsem).wait()
      if not first:
        wait_write(buf)
      posvecs = [pos_v[l, pl.ds(dg * 16, 16)] for dg in range(DG)]

      def jj_body(jj, car):
        for dg in range(DG):
          val = gbuf[jj, pl.ds(dg * 16, 16)] + posvecs[dg]
          plsc.store_scatter(tbuf, [rowvecs[dg] + jj], val)
        return car

      lax.fori_loop(0, BBLK, jj_body, 0)
      for g in range(8):
        pltpu.async_copy(tbuf.at[pl.ds(g * 1024, 1024)], out_hbm.at[l, g, c],
                         wsem)

    fire(bufs[0], bid0)
    fire(bufs[1], bid0 + 1)

    process(bufs[0], bid0, True)
    fire(bufs[0], bid0 + 2)
    process(bufs[1], bid0 + 1, True)
    fire(bufs[1], bid0 + 3)

    def pair_body(i, carry):
      bid = bid0 + 2 * i
      process(bufs[0], bid, False)
      fire(bufs[0], bid + 2)
      process(bufs[1], bid + 1, False)
      fire(bufs[1], bid + 3)
      return carry

    lax.fori_loop(1, BLOCKS_PER_W // 2 - 1, pair_body, 0)

    last = bid0 + BLOCKS_PER_W - 2
    process(bufs[0], last, False)
    process(bufs[1], last + 1, False)
    wait_write(bufs[0])
    wait_write(bufs[1])

  return emb_kernel


_emb = _build()


def kernel(inputs, token_table, pos_table):
  idxT = inputs.astype(jnp.int32).T
  W = _emb(idxT, token_table, pos_table)
  W5 = W.reshape(MAX_LEN, 8, NBC, 8, BBLK)
  return W5.transpose(2, 4, 0, 1, 3).reshape(BATCH, MAX_LEN, EMBED_DIM)


# speedup vs baseline: 1.4020x; 1.4020x over previous
import functools

import jax
import jax.numpy as jnp
from jax import lax
from jax.experimental import pallas as pl
from jax.experimental.pallas import tpu as pltpu
from jax.experimental.pallas import tpu_sc as plsc

VOCAB = 1000000
MAX_LEN = 200
EMBED_DIM = 64
BATCH = 4096

NUM_CORES = 2
NUM_SUBCORES = 16
NW = NUM_CORES * NUM_SUBCORES
BBLK = 128
NBC = BATCH // BBLK
N_BLOCKS = MAX_LEN * NBC
BLOCKS_PER_W = N_BLOCKS // NW
IDX_PER_W = BLOCKS_PER_W * BBLK
DG = EMBED_DIM // 16


def _build():
  mesh = plsc.VectorSubcoreMesh(core_axis_name="c", subcore_axis_name="s")

  @functools.partial(
      pl.kernel,
      mesh=mesh,
      compiler_params=pltpu.CompilerParams(use_tc_tiling_on_sc=False,
                                           needs_layout_passes=False),
      out_type=jax.ShapeDtypeStruct((MAX_LEN, 8, NBC, 8 * BBLK), jnp.float32),
      scratch_types=[
          pltpu.VMEM((IDX_PER_W,), jnp.int32),
          pltpu.VMEM((BBLK, EMBED_DIM), jnp.float32),
          pltpu.VMEM((BBLK, EMBED_DIM), jnp.float32),
          pltpu.VMEM((EMBED_DIM * BBLK,), jnp.float32),
          pltpu.VMEM((EMBED_DIM * BBLK,), jnp.float32),
          pltpu.VMEM((MAX_LEN, EMBED_DIM), jnp.float32),
          pltpu.SemaphoreType.DMA,
          pltpu.SemaphoreType.DMA,
          pltpu.SemaphoreType.DMA,
          pltpu.SemaphoreType.DMA,
      ],
  )
  def emb_kernel(idx_hbm, table_hbm, pos_hbm, out_hbm,
                 idx_all, g_a, g_b, t_a, t_b, pos_v,
                 gsem_a, gsem_b, wsem_a, wsem_b):
    wid = lax.axis_index("s") * NUM_CORES + lax.axis_index("c")
    bid0 = wid * BLOCKS_PER_W

    pltpu.sync_copy(idx_hbm.at[pl.ds(wid * IDX_PER_W, IDX_PER_W)], idx_all)
    pltpu.sync_copy(pos_hbm, pos_v)

    iota16 = lax.iota(jnp.int32, 16)
    rowvecs = [(dg * 16 + iota16) * BBLK for dg in range(DG)]

    bufs = ((g_a, t_a, gsem_a, wsem_a),
            (g_b, t_b, gsem_b, wsem_b))

    def fire(buf, i):
      gbuf, _, gsem, _ = buf
      pltpu.async_copy(table_hbm.at[idx_all.at[pl.ds(i * BBLK, BBLK)]],
                       gbuf, gsem)

    def wait_write(buf):
      _, tbuf, _, wsem = buf
      for g in range(8):
        pltpu.make_async_copy(tbuf.at[pl.ds(g * 1024, 1024)],
                              out_hbm.at[0, g, 0], wsem).wait()

    def process(buf, i, first):
      gbuf, tbuf, gsem, wsem = buf
      bid = bid0 + i
      l = bid // NBC
      c = bid - l * NBC
      pltpu.make_async_copy(table_hbm.at[idx_all.at[pl.ds(i * BBLK, BBLK)]],
                            gbuf, gsem).wait()
      if not first:
        wait_write(buf)
      posvecs = [pos_v[l, pl.ds(dg * 16, 16)] for dg in range(DG)]

      @plsc.parallel_loop(0, BBLK, unroll=8)
      def jj_body(jj):
        for dg in range(DG):
          val = gbuf[jj, pl.ds(dg * 16, 16)] + posvecs[dg]
          plsc.store_scatter(tbuf, [rowvecs[dg] + jj], val)

      for g in range(8):
        pltpu.async_copy(tbuf.at[pl.ds(g * 1024, 1024)], out_hbm.at[l, g, c],
                         wsem)

    fire(bufs[0], 0)
    fire(bufs[1], 1)

    process(bufs[0], 0, True)
    fire(bufs[0], 2)
    process(bufs[1], 1, True)
    fire(bufs[1], 3)

    def pair_body(i, carry):
      k = 2 * i
      process(bufs[0], k, False)
      fire(bufs[0], k + 2)
      process(bufs[1], k + 1, False)
      fire(bufs[1], k + 3)
      return carry

    lax.fori_loop(1, BLOCKS_PER_W // 2 - 1, pair_body, 0)

    last = BLOCKS_PER_W - 2
    process(bufs[0], last, False)
    process(bufs[1], last + 1, False)
    wait_write(bufs[0])
    wait_write(bufs[1])

  return emb_kernel


_emb = _build()


def kernel(inputs, token_table, pos_table):
  idx_flat = inputs.astype(jnp.int32).T.reshape(-1)
  W = _emb(idx_flat, token_table, pos_table)
  W5 = W.reshape(MAX_LEN, 8, NBC, 8, BBLK)
  return W5.transpose(2, 4, 0, 1, 3).reshape(BATCH, MAX_LEN, EMBED_DIM)
